# baseline (speedup 1.0000x reference)
import jax
import jax.numpy as jnp
from jax import lax
from jax.experimental import pallas as pl
from jax.experimental.pallas import tpu as pltpu
from jax.experimental.pallas import tpu_sc as plsc

B = 16384
L = 50
D = 64
NC = 2
NS = 16
NW = NC * NS
BPW = B // NW
CB = 8
NCH = BPW // CB
CROWS = CB * L
PROWS = CROWS + 16
IDXR = 100
CIR = CROWS // IDXR
RPW = 256
_IDX_GROUPS = [(o, False) for o in (0, 16, 32, 48, 64, 80)] + [(84, True)]


def _sc_scores_body(tgt_hbm, ctx_hbm, temb_hbm, cemb_hbm, scores_hbm,
                    tidx_v, tpair_v, tpar_v, trow0, trow1, cidx_v,
                    cpair0, cpair1, cpar0, cpar1, crow0, crow1, score_v,
                    sem_t, sem_c0, sem_c1):
    w = lax.axis_index("s") * NC + lax.axis_index("c")

    lane = lax.iota(jnp.int32, 16)
    zeros16 = jnp.zeros((16,), jnp.float32)
    tail_mask = lane >= 12

    cpar0[pl.ds(CROWS, 16)] = jnp.zeros((16,), jnp.int32)
    cpar1[pl.ds(CROWS, 16)] = jnp.zeros((16,), jnp.int32)

    def drain(dummy_src, dst, sem):
        pltpu.make_async_copy(dummy_src, dst, sem).wait()

    pltpu.sync_copy(tgt_hbm.at[pl.ds(w * 4, 4)], tidx_v)
    for k in range(4):
        for g in range(8):
            iv = tidx_v[k, pl.ds(g * 16, 16)]
            tpair_v[pl.ds(k * 128 + g * 16, 16)] = lax.shift_right_logical(iv, 1)
            tpar_v[pl.ds(k * 128 + g * 16, 16)] = jnp.bitwise_and(iv, 1)

    def stage_issue(ch, cpair, cpar, crow, trow, sem):
        off = w * RPW + jnp.minimum(ch, NCH - 1) * CIR
        pltpu.sync_copy(ctx_hbm.at[pl.ds(off, CIR)], cidx_v)
        for k in range(CIR):
            kvec = jnp.full((16,), k, jnp.int32)
            for goff, tail in _IDX_GROUPS:
                iv = cidx_v[k, pl.ds(goff, 16)]
                pair = lax.shift_right_logical(iv, 1)
                par = jnp.bitwise_and(iv, 1)
                mask = tail_mask if tail else None
                gidx = goff + lane
                plsc.store_scatter(cpair, [kvec, gidx], pair, mask=mask)
                plsc.store_scatter(cpar, [k * IDXR + gidx], par, mask=mask)
        for k in range(CIR):
            pltpu.async_copy(cemb_hbm.at[cpair.at[k]],
                             crow.at[pl.ds(k * IDXR, IDXR)], sem)
        pltpu.async_copy(
            temb_hbm.at[tpair_v.at[pl.ds(jnp.minimum(ch, NCH - 1) * CB, CB)]],
            trow, sem)

    def wait_chunk(crow, trow, sem):
        drain(cemb_hbm.at[pl.ds(0, CROWS)], crow.at[pl.ds(0, CROWS)], sem)
        drain(temb_hbm.at[pl.ds(0, CB)], trow, sem)

    def compute(crow, cpar, trow, ch):
        def bb(b, c):
            wb = ch * CB + b
            pvb = plsc.load_gather(tpar_v, [jnp.full((16,), wb, jnp.int32)])
            tps = pvb[0] * 64
            trs = [trow[b, pl.ds(tps + 16 * k, 16)] for k in range(4)]
            slot = []
            ccol = []
            acc = []
            for g in range(4):
                sv = b * L + g * 16 + lane
                slot.append(sv)
                cpv = plsc.load_gather(cpar, [sv])
                ccol.append(cpv * 64)
                acc.append(zeros16)
            for d in range(D):
                tv = jnp.full((16,), trs[d // 16][d % 16])
                for g in range(4):
                    cv = plsc.load_gather(crow, [slot[g], ccol[g] + d])
                    acc[g] = acc[g] + tv * cv
            for g in range(4):
                score_v[b, pl.ds(g * 16, 16)] = acc[g]
            return c
        lax.fori_loop(0, CB, bb, 0)
        pltpu.sync_copy(score_v,
                        scores_hbm.at[pl.ds(w * BPW + ch * CB, CB)])

    stage_issue(jnp.int32(0), cpair0, cpar0, crow0, trow0, sem_c0)

    def two_chunks(i, c):
        ch = i * 2
        stage_issue(ch + 1, cpair1, cpar1, crow1, trow1, sem_c1)
        wait_chunk(crow0, trow0, sem_c0)
        compute(crow0, cpar0, trow0, ch)
        stage_issue(ch + 2, cpair0, cpar0, crow0, trow0, sem_c0)
        wait_chunk(crow1, trow1, sem_c1)
        compute(crow1, cpar1, trow1, ch + 1)
        return c

    lax.fori_loop(0, NCH // 2, two_chunks, 0)

    wait_chunk(crow0, trow0, sem_c0)


def _sc_scores(tgt2, ctx2, temb2, cemb2):
    mesh = plsc.VectorSubcoreMesh(core_axis_name="c", subcore_axis_name="s")
    return pl.kernel(
        _sc_scores_body,
        out_type=jax.ShapeDtypeStruct((B, 64), jnp.float32),
        mesh=mesh,
        compiler_params=pltpu.CompilerParams(needs_layout_passes=False),
        scratch_types=[
            pltpu.VMEM((4, 128), jnp.int32),
            pltpu.VMEM((BPW,), jnp.int32),
            pltpu.VMEM((BPW,), jnp.int32),
            pltpu.VMEM((CB, 128), jnp.float32),
            pltpu.VMEM((CB, 128), jnp.float32),
            pltpu.VMEM((CIR, IDXR), jnp.int32),
            pltpu.VMEM((CIR, IDXR), jnp.int32),
            pltpu.VMEM((CIR, IDXR), jnp.int32),
            pltpu.VMEM((PROWS,), jnp.int32),
            pltpu.VMEM((PROWS,), jnp.int32),
            pltpu.VMEM((PROWS, 128), jnp.float32),
            pltpu.VMEM((PROWS, 128), jnp.float32),
            pltpu.VMEM((CB, 64), jnp.float32),
            pltpu.SemaphoreType.DMA,
            pltpu.SemaphoreType.DMA,
            pltpu.SemaphoreType.DMA,
        ],
    )(tgt2, ctx2, temb2, cemb2)


def _reduce_body(s_ref, o_ref):
    x = s_ref[...]
    col = lax.broadcasted_iota(jnp.int32, x.shape, 1)
    t = jnp.maximum(-x, 0.0) + jnp.log1p(jnp.exp(-jnp.abs(x)))
    o_ref[0, 0] = jnp.sum(jnp.where(col < L, t, 0.0)) * (1.0 / (B * L))


def _reduce(scores):
    return pl.pallas_call(
        _reduce_body,
        out_shape=jax.ShapeDtypeStruct((1, 1), jnp.float32),
        out_specs=pl.BlockSpec(memory_space=pltpu.SMEM),
    )(scores)


def kernel(target, context, target_embeddings, context_embeddings):
    tgt2 = target.reshape(B // 128, 128)
    ctx2 = context.reshape(B * L // IDXR, IDXR)
    temb2 = target_embeddings.reshape(500000, 128)
    cemb2 = context_embeddings.reshape(500000, 128)
    scores = _sc_scores(tgt2, ctx2, temb2, cemb2)
    return _reduce(scores)[0, 0]


# speedup vs baseline: 1.3923x; 1.3923x over previous
import jax
import jax.numpy as jnp
from jax import lax
from jax.experimental import pallas as pl
from jax.experimental.pallas import tpu as pltpu
from jax.experimental.pallas import tpu_sc as plsc

B = 16384
L = 50
D = 64
NC = 2
NS = 16
NW = NC * NS
BPW = B // NW
CB = 4
NCH = BPW // CB
CROWS = CB * L
IDXR = 100
CIR = CROWS // IDXR
RPW = 256


_IDX_GROUPS = [(o, tuple(range(16))) for o in (0, 16, 32, 48, 64, 80)] + [
    (84, (12, 13, 14, 15))]


def _sc_scores_body(tgt_hbm, ctx_hbm, temb_hbm, cemb_hbm, scores_hbm,
                    tidx_v, trow_v, cidx_v, crow0, crow1, score_v,
                    sem_t, sem_c0, sem_c1):
    w = lax.axis_index("s") * NC + lax.axis_index("c")

    lane = lax.iota(jnp.int32, 16)
    m15 = lane == 15
    zeros16 = jnp.zeros((16,), jnp.float32)

    for r in range(CB):
        score_v[r, pl.ds(48, 16)] = zeros16

    def drain(dummy_src, dst, sem):
        pltpu.make_async_copy(dummy_src, dst, sem).wait()

    def issue_rows(idx_ref, nrows, rowlen, groups, table_hbm, dst_ref, sem):
        def go(k, c):
            for off, lanes in groups:
                iv = idx_ref[k, pl.ds(off, 16)]
                idxs = [(u, iv[u]) for u in lanes]
                for u, idx in idxs:
                    pltpu.async_copy(
                        table_hbm.at[pl.ds(idx, 1)],
                        dst_ref.at[pl.ds(k * rowlen + off + u, 1)], sem)
            return c
        lax.fori_loop(0, nrows, go, 0)

    _TGT_GROUPS = [(o, tuple(range(16))) for o in range(0, 128, 16)]

    pltpu.sync_copy(tgt_hbm.at[pl.ds(w * 4, 4)], tidx_v)
    issue_rows(tidx_v, 4, 128, _TGT_GROUPS, temb_hbm, trow_v, sem_t)
    drain(temb_hbm.at[pl.ds(0, BPW)], trow_v, sem_t)

    def idx_copy(ch):
        off = w * RPW + jnp.minimum(ch, NCH - 1) * CIR
        pltpu.sync_copy(ctx_hbm.at[pl.ds(off, CIR)], cidx_v)

    def crow_drain(crow, sem):
        drain(cemb_hbm.at[pl.ds(0, CROWS)], crow, sem)

    def _batch_groups(b):
        s0 = b * L
        k, off0 = s0 // IDXR, s0 % IDXR
        if off0 == 0:
            return [(k, 0, range(16)), (k, 16, range(16)),
                    (k, 32, range(16)), (k, 48, range(2))]
        return [(k, 48, range(2, 16)), (k, 64, range(16)),
                (k, 80, range(16)), (k, 84, range(12, 16))]

    def compute_issue(crow, ch, crow_next, sem_next):
        for b in range(CB):
            for k, off, lanes in _batch_groups(b):
                iv = cidx_v[k, pl.ds(off, 16)]
                for u in lanes:
                    pltpu.async_copy(
                        cemb_hbm.at[pl.ds(iv[u], 1)],
                        crow_next.at[pl.ds(k * IDXR + off + u, 1)], sem_next)
            wb = ch * CB + b
            t0 = trow_v[wb, pl.ds(0, 16)]
            t1 = trow_v[wb, pl.ds(16, 16)]
            t2 = trow_v[wb, pl.ds(32, 16)]
            t3 = trow_v[wb, pl.ds(48, 16)]
            bvec = jnp.full((16,), b, jnp.int32)
            rb = b * L
            for l in range(L):
                r = rb + l
                c0 = crow[r, pl.ds(0, 16)]
                c1 = crow[r, pl.ds(16, 16)]
                c2 = crow[r, pl.ds(32, 16)]
                c3 = crow[r, pl.ds(48, 16)]
                m = (t0 * c0 + t1 * c1) + (t2 * c2 + t3 * c3)
                s = plsc.cumsum(m)
                plsc.store_scatter(
                    score_v, [bvec, jnp.full((16,), l, jnp.int32)],
                    s, mask=m15)
        pltpu.sync_copy(score_v,
                        scores_hbm.at[pl.ds(w * BPW + ch * CB, CB)])

    idx_copy(jnp.int32(0))
    issue_rows(cidx_v, CIR, IDXR, _IDX_GROUPS, cemb_hbm, crow0, sem_c0)

    def two_chunks(i, c):
        ch = i * 2
        idx_copy(ch + 1)
        crow_drain(crow0, sem_c0)
        compute_issue(crow0, ch, crow1, sem_c1)
        idx_copy(ch + 2)
        crow_drain(crow1, sem_c1)
        compute_issue(crow1, ch + 1, crow0, sem_c0)
        return c

    lax.fori_loop(0, NCH // 2, two_chunks, 0)

    crow_drain(crow0, sem_c0)


def _sc_scores(tgt2, ctx2, temb, cemb):
    mesh = plsc.VectorSubcoreMesh(core_axis_name="c", subcore_axis_name="s")
    return pl.kernel(
        _sc_scores_body,
        out_type=jax.ShapeDtypeStruct((B, 64), jnp.float32),
        mesh=mesh,
        compiler_params=pltpu.CompilerParams(needs_layout_passes=False),
        scratch_types=[
            pltpu.VMEM((4, 128), jnp.int32),
            pltpu.VMEM((BPW, D), jnp.float32),
            pltpu.VMEM((CIR, IDXR), jnp.int32),
            pltpu.VMEM((CROWS, D), jnp.float32),
            pltpu.VMEM((CROWS, D), jnp.float32),
            pltpu.VMEM((CB, 64), jnp.float32),
            pltpu.SemaphoreType.DMA,
            pltpu.SemaphoreType.DMA,
            pltpu.SemaphoreType.DMA,
        ],
    )(tgt2, ctx2, temb, cemb)


def _reduce_body(s_ref, o_ref):
    x = s_ref[...]
    col = lax.broadcasted_iota(jnp.int32, x.shape, 1)
    t = jnp.maximum(-x, 0.0) + jnp.log1p(jnp.exp(-jnp.abs(x)))
    o_ref[0, 0] = jnp.sum(jnp.where(col < L, t, 0.0)) * (1.0 / (B * L))


def _reduce(scores):
    return pl.pallas_call(
        _reduce_body,
        out_shape=jax.ShapeDtypeStruct((1, 1), jnp.float32),
        out_specs=pl.BlockSpec(memory_space=pltpu.SMEM),
    )(scores)


def kernel(target, context, target_embeddings, context_embeddings):
    tgt2 = target.reshape(B // 128, 128)
    ctx2 = context.reshape(B * L // IDXR, IDXR)
    scores = _sc_scores(tgt2, ctx2, target_embeddings, context_embeddings)
    return _reduce(scores)[0, 0]


# speedup vs baseline: 1.4265x; 1.0245x over previous
import jax
import jax.numpy as jnp
from jax import lax
from jax.experimental import pallas as pl
from jax.experimental.pallas import tpu as pltpu
from jax.experimental.pallas import tpu_sc as plsc

B = 16384
L = 50
D = 64
NC = 2
NS = 16
NW = NC * NS
BPW = B // NW
CB = 4
NCH = BPW // CB
CROWS = CB * L
IDXR = 100
CIR = CROWS // IDXR
RPW = 256
_IDX_GROUPS = (0, 16, 32, 48, 64, 80, 84)
_IDX_ROW_BYTES_ROWS = 7 * 16


def _sc_scores_body(tgt_hbm, ctx_hbm, temb_hbm, cemb_hbm, scores_hbm,
                    tidx_v, trow_v, cidx_v, crow0, crow1, score_v,
                    sem_t, sem_c0, sem_c1):
    w = lax.axis_index("s") * NC + lax.axis_index("c")

    lane = lax.iota(jnp.int32, 16)
    m15 = lane == 15
    zeros16 = jnp.zeros((16,), jnp.float32)

    for r in range(CB):
        score_v[r, pl.ds(48, 16)] = zeros16

    def drain(dummy_src, dst, sem):
        pltpu.make_async_copy(dummy_src, dst, sem).wait()

    def issue_rows(idx_ref, nrows, rowlen, offs, table_hbm, dst_ref, sem):
        def go(k, c):
            for off in offs:
                iv = idx_ref[k, pl.ds(off, 16)]
                pltpu.async_copy(
                    table_hbm.at[iv],
                    dst_ref.at[pl.ds(k * rowlen + off, 16)], sem)
            return c
        lax.fori_loop(0, nrows, go, 0)

    _TGT_GROUPS = tuple(range(0, 128, 16))

    pltpu.sync_copy(tgt_hbm.at[pl.ds(w * 4, 4)], tidx_v)
    issue_rows(tidx_v, 4, 128, _TGT_GROUPS, temb_hbm, trow_v, sem_t)
    drain(temb_hbm.at[pl.ds(0, BPW)], trow_v, sem_t)

    def idx_copy(ch):
        off = w * RPW + jnp.minimum(ch, NCH - 1) * CIR
        pltpu.sync_copy(ctx_hbm.at[pl.ds(off, CIR)], cidx_v)

    def crow_drain(crow, sem):
        extra = CIR * _IDX_ROW_BYTES_ROWS - CROWS
        drain(cemb_hbm.at[pl.ds(0, CROWS)], crow, sem)
        drain(cemb_hbm.at[pl.ds(0, extra)], crow.at[pl.ds(0, extra)], sem)

    def compute(crow, ch):
        def bb(b, c):
            wb = ch * CB + b
            t0 = trow_v[wb, pl.ds(0, 16)]
            t1 = trow_v[wb, pl.ds(16, 16)]
            t2 = trow_v[wb, pl.ds(32, 16)]
            t3 = trow_v[wb, pl.ds(48, 16)]
            bvec = jnp.full((16,), b, jnp.int32)
            rb = b * L
            for l in range(L):
                r = rb + l
                c0 = crow[r, pl.ds(0, 16)]
                c1 = crow[r, pl.ds(16, 16)]
                c2 = crow[r, pl.ds(32, 16)]
                c3 = crow[r, pl.ds(48, 16)]
                m = (t0 * c0 + t1 * c1) + (t2 * c2 + t3 * c3)
                s = plsc.cumsum(m)
                plsc.store_scatter(
                    score_v, [bvec, jnp.full((16,), l, jnp.int32)],
                    s, mask=m15)
            return c
        lax.fori_loop(0, CB, bb, 0)
        pltpu.sync_copy(score_v,
                        scores_hbm.at[pl.ds(w * BPW + ch * CB, CB)])

    idx_copy(jnp.int32(0))
    issue_rows(cidx_v, CIR, IDXR, _IDX_GROUPS, cemb_hbm, crow0, sem_c0)

    def two_chunks(i, c):
        ch = i * 2
        idx_copy(ch + 1)
        issue_rows(cidx_v, CIR, IDXR, _IDX_GROUPS, cemb_hbm, crow1, sem_c1)
        crow_drain(crow0, sem_c0)
        compute(crow0, ch)
        idx_copy(ch + 2)
        issue_rows(cidx_v, CIR, IDXR, _IDX_GROUPS, cemb_hbm, crow0, sem_c0)
        crow_drain(crow1, sem_c1)
        compute(crow1, ch + 1)
        return c

    lax.fori_loop(0, NCH // 2, two_chunks, 0)

    crow_drain(crow0, sem_c0)


def _sc_scores(tgt2, ctx2, temb, cemb):
    mesh = plsc.VectorSubcoreMesh(core_axis_name="c", subcore_axis_name="s")
    return pl.kernel(
        _sc_scores_body,
        out_type=jax.ShapeDtypeStruct((B, 64), jnp.float32),
        mesh=mesh,
        compiler_params=pltpu.CompilerParams(needs_layout_passes=False,
                                             use_tc_tiling_on_sc=False),
        scratch_types=[
            pltpu.VMEM((4, 128), jnp.int32),
            pltpu.VMEM((BPW, D), jnp.float32),
            pltpu.VMEM((CIR, IDXR), jnp.int32),
            pltpu.VMEM((CROWS, D), jnp.float32),
            pltpu.VMEM((CROWS, D), jnp.float32),
            pltpu.VMEM((CB, 64), jnp.float32),
            pltpu.SemaphoreType.DMA,
            pltpu.SemaphoreType.DMA,
            pltpu.SemaphoreType.DMA,
        ],
    )(tgt2, ctx2, temb, cemb)


def _reduce_body(s_ref, o_ref):
    x = s_ref[...]
    col = lax.broadcasted_iota(jnp.int32, x.shape, 1)
    t = jnp.maximum(-x, 0.0) + jnp.log1p(jnp.exp(-jnp.abs(x)))
    o_ref[0, 0] = jnp.sum(jnp.where(col < L, t, 0.0)) * (1.0 / (B * L))


def _reduce(scores):
    return pl.pallas_call(
        _reduce_body,
        out_shape=jax.ShapeDtypeStruct((1, 1), jnp.float32),
        out_specs=pl.BlockSpec(memory_space=pltpu.SMEM),
    )(scores)


def kernel(target, context, target_embeddings, context_embeddings):
    tgt2 = target.reshape(B // 128, 128)
    ctx2 = context.reshape(B * L // IDXR, IDXR)
    scores = _sc_scores(tgt2, ctx2, target_embeddings, context_embeddings)
    return _reduce(scores)[0, 0]


# speedup vs baseline: 1.7526x; 1.2286x over previous
import jax
import jax.numpy as jnp
from jax import lax
from jax.experimental import pallas as pl
from jax.experimental.pallas import tpu as pltpu
from jax.experimental.pallas import tpu_sc as plsc

B = 16384
L = 50
D = 64
NC = 2
NS = 16
NW = NC * NS
BPW = B // NW
CB = 4
NCH = BPW // CB
CROWS = CB * L
IDXR = 100
CIR = CROWS // IDXR
RPW = 256
_IDX_GROUPS = [(o, tuple(range(16))) for o in (0, 16, 32, 48, 64, 80)] + [
    (84, (12, 13, 14, 15))]


def _sc_scores_body(tgt_hbm, ctx_hbm, temb_hbm, cemb_hbm, scores_hbm,
                    tidx_v, trow_v, cidx_v, crow0, crow1, score_v,
                    sem_t, sem_c0, sem_c1):
    w = lax.axis_index("s") * NC + lax.axis_index("c")

    lane = lax.iota(jnp.int32, 16)
    m15 = lane == 15
    zeros16 = jnp.zeros((16,), jnp.float32)

    for r in range(CB):
        score_v[r, pl.ds(48, 16)] = zeros16

    def drain(dummy_src, dst, sem):
        pltpu.make_async_copy(dummy_src, dst, sem).wait()

    def issue_rows(idx_ref, nrows, rowlen, groups, table_hbm, dst_ref, sem):
        def go(k, c):
            for off, lanes in groups:
                iv = idx_ref[k, pl.ds(off, 16)]
                for u in lanes:
                    slot = off + u
                    pltpu.async_copy(
                        table_hbm.at[pl.ds(iv[u], 1)],
                        dst_ref.at[pl.ds(k * rowlen + slot, 1)], sem)
            return c
        lax.fori_loop(0, nrows, go, 0)

    _TGT_GROUPS = [(o, tuple(range(16))) for o in range(0, 128, 16)]

    pltpu.sync_copy(tgt_hbm.at[pl.ds(w * 4, 4)], tidx_v)
    issue_rows(tidx_v, 4, 128, _TGT_GROUPS, temb_hbm, trow_v, sem_t)
    drain(temb_hbm.at[pl.ds(0, BPW)], trow_v, sem_t)

    def idx_copy(ch):
        off = w * RPW + jnp.minimum(ch, NCH - 1) * CIR
        pltpu.sync_copy(ctx_hbm.at[pl.ds(off, CIR)], cidx_v)

    def crow_drain(crow, sem):
        drain(cemb_hbm.at[pl.ds(0, CROWS)], crow, sem)

    def compute(crow, ch):
        def bb(b, c):
            wb = ch * CB + b
            t0 = trow_v[wb, pl.ds(0, 16)]
            t1 = trow_v[wb, pl.ds(16, 16)]
            t2 = trow_v[wb, pl.ds(32, 16)]
            t3 = trow_v[wb, pl.ds(48, 16)]
            bvec = jnp.full((16,), b, jnp.int32)
            rb = b * L
            for l in range(L):
                r = rb + l
                c0 = crow[r, pl.ds(0, 16)]
                c1 = crow[r, pl.ds(16, 16)]
                c2 = crow[r, pl.ds(32, 16)]
                c3 = crow[r, pl.ds(48, 16)]
                m = (t0 * c0 + t1 * c1) + (t2 * c2 + t3 * c3)
                s = plsc.cumsum(m)
                plsc.store_scatter(
                    score_v, [bvec, jnp.full((16,), l, jnp.int32)],
                    s, mask=m15)
            return c
        lax.fori_loop(0, CB, bb, 0)
        pltpu.sync_copy(score_v,
                        scores_hbm.at[pl.ds(w * BPW + ch * CB, CB)])

    idx_copy(jnp.int32(0))
    issue_rows(cidx_v, CIR, IDXR, _IDX_GROUPS, cemb_hbm, crow0, sem_c0)

    def two_chunks(i, c):
        ch = i * 2
        idx_copy(ch + 1)
        issue_rows(cidx_v, CIR, IDXR, _IDX_GROUPS, cemb_hbm, crow1, sem_c1)
        crow_drain(crow0, sem_c0)
        compute(crow0, ch)
        idx_copy(ch + 2)
        issue_rows(cidx_v, CIR, IDXR, _IDX_GROUPS, cemb_hbm, crow0, sem_c0)
        crow_drain(crow1, sem_c1)
        compute(crow1, ch + 1)
        return c

    lax.fori_loop(0, NCH // 2, two_chunks, 0)

    crow_drain(crow0, sem_c0)


def _sc_scores(tgt2, ctx2, temb, cemb):
    mesh = plsc.VectorSubcoreMesh(core_axis_name="c", subcore_axis_name="s")
    return pl.kernel(
        _sc_scores_body,
        out_type=jax.ShapeDtypeStruct((B, 64), jnp.float32),
        mesh=mesh,
        compiler_params=pltpu.CompilerParams(needs_layout_passes=False),
        scratch_types=[
            pltpu.VMEM((4, 128), jnp.int32),
            pltpu.VMEM((BPW, D), jnp.float32),
            pltpu.VMEM((CIR, IDXR), jnp.int32),
            pltpu.VMEM((CROWS, D), jnp.float32),
            pltpu.VMEM((CROWS, D), jnp.float32),
            pltpu.VMEM((CB, 64), jnp.float32),
            pltpu.SemaphoreType.DMA,
            pltpu.SemaphoreType.DMA,
            pltpu.SemaphoreType.DMA,
        ],
    )(tgt2, ctx2, temb, cemb)


def _reduce_body(s_ref, o_ref):
    x = s_ref[...]
    col = lax.broadcasted_iota(jnp.int32, x.shape, 1)
    t = jnp.maximum(-x, 0.0) + jnp.log1p(jnp.exp(-jnp.abs(x)))
    o_ref[0, 0] = jnp.sum(jnp.where(col < L, t, 0.0)) * (1.0 / (B * L))


def _reduce(scores):
    return pl.pallas_call(
        _reduce_body,
        out_shape=jax.ShapeDtypeStruct((1, 1), jnp.float32),
        out_specs=pl.BlockSpec(memory_space=pltpu.SMEM),
    )(scores)


def kernel(target, context, target_embeddings, context_embeddings):
    tgt2 = target.reshape(B // 128, 128)
    ctx2 = context.reshape(B * L // IDXR, IDXR)
    scores = _sc_scores(tgt2, ctx2, target_embeddings, context_embeddings)
    return _reduce(scores)[0, 0]
